# Initial kernel scaffold; baseline (speedup 1.0000x reference)
import jax
import jax.numpy as jnp
from jax.experimental import pallas as pl


def kernel(x, edge_index):
    raise NotImplementedError("write your pallas kernel here")


# speedup vs baseline: 30.6618x; 30.6618x over previous
import jax
import jax.numpy as jnp
from jax import lax
from jax.experimental import pallas as pl
from jax.experimental.pallas import tpu as pltpu
from jax.experimental.pallas import tpu_sc as plsc

N_NODES = 10000
N_EDGES = 160000
D_FEAT = 256

NC, NS, L = 2, 16, 16
NPAD = 10240
NODES_PT = NPAD // NS
EH = N_EDGES // NS
EV = N_EDGES // (NC * NS)
EVP = EV + 8


def _rowsum_body(x_ref, o_ref):
    o_ref[...] = jnp.sum(x_ref[...], axis=1)[None, None, :]


def _rsqrt16(d):
    bits = lax.bitcast_convert_type(d, jnp.int32)
    y = lax.bitcast_convert_type(jnp.int32(0x5F3759DF) - (bits >> 1), jnp.float32)
    hd = 0.5 * d
    for _ in range(3):
        y = y * (1.5 - hd * y * y)
    return jnp.where(d > 0.5, y, 0.0)


def _sc_body(rowsum, efrom, eto, ones_h, out,
             toh_v, ones_v, from_v, to_v, sf_v, dt_v, outb_v,
             deg_v, dis_v, s_v, rs_v, zero_v,
             hist_sh, dis_sh, s_sh, sem1, sem2):
    c = lax.axis_index("c")
    s = lax.axis_index("s")
    node0 = s * NODES_PT

    def zb(i, carry):
        zero_v[pl.ds(i * L, L)] = jnp.zeros((L,), jnp.float32)
        return carry
    lax.fori_loop(0, NODES_PT // L, zb, None)
    pltpu.sync_copy(zero_v, hist_sh.at[pl.ds(node0, NODES_PT)])
    pltpu.sync_copy(eto.at[pl.ds(s * EH, EH)], toh_v)
    pltpu.sync_copy(ones_h, ones_v)
    plsc.subcore_barrier()

    pltpu.sync_copy(ones_v, hist_sh.at[toh_v], add=True)
    plsc.subcore_barrier()

    pltpu.sync_copy(hist_sh.at[pl.ds(node0, NODES_PT)], deg_v)
    pltpu.sync_copy(rowsum.at[pl.ds(node0, NODES_PT)], rs_v)

    def cb(i, carry):
        sl = pl.ds(i * L, L)
        dis = _rsqrt16(deg_v[sl])
        dis_v[sl] = dis
        s_v[sl] = dis * rs_v[sl]
        return carry
    lax.fori_loop(0, NODES_PT // L, cb, None)
    pltpu.sync_copy(dis_v, dis_sh.at[pl.ds(node0, NODES_PT)])
    pltpu.sync_copy(s_v, s_sh.at[pl.ds(node0, NODES_PT)])
    plsc.subcore_barrier()

    base = c * (N_EDGES // NC) + s * EV
    pltpu.sync_copy(efrom.at[pl.ds(base, EV)], from_v.at[pl.ds(0, EV)])
    pltpu.sync_copy(eto.at[pl.ds(base, EV)], to_v.at[pl.ds(0, EV)])
    lanes = lax.iota(jnp.int32, L)
    nvalid = EV - (EVP - L)
    tl = pl.ds(EVP - L, L)
    from_v[tl] = jnp.where(lanes < nvalid, from_v[tl], 0)
    to_v[tl] = jnp.where(lanes < nvalid, to_v[tl], 0)
    cp1 = pltpu.async_copy(s_sh.at[from_v], sf_v, sem1)
    cp2 = pltpu.async_copy(dis_sh.at[to_v], dt_v, sem2)
    cp1.wait()
    cp2.wait()

    def eb(i, carry):
        sl = pl.ds(i * L, L)
        outb_v[sl] = sf_v[sl] * dt_v[sl]
        return carry
    lax.fori_loop(0, EVP // L, eb, None)
    pltpu.sync_copy(outb_v.at[pl.ds(0, EV)], out.at[pl.ds(base, EV)])


def kernel(x, edge_index):
    x = x.astype(jnp.float32)
    ei = edge_index.astype(jnp.int32)

    gridr = (N_NODES + 127) // 128
    rowsum2d = pl.pallas_call(
        _rowsum_body,
        grid=(gridr,),
        in_specs=[pl.BlockSpec((128, D_FEAT), lambda i: (i, 0))],
        out_specs=pl.BlockSpec((1, 1, 128), lambda i: (i, 0, 0)),
        out_shape=jax.ShapeDtypeStruct((gridr, 1, 128), jnp.float32),
    )(x)
    rs_pad = jnp.zeros((NPAD,), jnp.float32).at[:N_NODES].set(
        rowsum2d.reshape(-1)[:N_NODES])
    ones_h = jnp.ones((EH,), jnp.float32)

    mesh = plsc.VectorSubcoreMesh(core_axis_name="c", subcore_axis_name="s")
    sc = pl.kernel(
        _sc_body,
        out_type=jax.ShapeDtypeStruct((N_EDGES,), jnp.float32),
        mesh=mesh,
        scratch_types=[
            pltpu.VMEM((EH,), jnp.int32),
            pltpu.VMEM((EH,), jnp.float32),
            pltpu.VMEM((EVP,), jnp.int32),
            pltpu.VMEM((EVP,), jnp.int32),
            pltpu.VMEM((EVP,), jnp.float32),
            pltpu.VMEM((EVP,), jnp.float32),
            pltpu.VMEM((EVP,), jnp.float32),
            pltpu.VMEM((NODES_PT,), jnp.float32),
            pltpu.VMEM((NODES_PT,), jnp.float32),
            pltpu.VMEM((NODES_PT,), jnp.float32),
            pltpu.VMEM((NODES_PT,), jnp.float32),
            pltpu.VMEM((NODES_PT,), jnp.float32),
            pltpu.VMEM_SHARED((NPAD,), jnp.float32),
            pltpu.VMEM_SHARED((NPAD,), jnp.float32),
            pltpu.VMEM_SHARED((NPAD,), jnp.float32),
            pltpu.SemaphoreType.DMA,
            pltpu.SemaphoreType.DMA,
        ],
    )
    return sc(rs_pad, ei[0], ei[1], ones_h)


# speedup vs baseline: 32.5088x; 1.0602x over previous
import jax
import jax.numpy as jnp
from jax import lax
from jax.experimental import pallas as pl
from jax.experimental.pallas import tpu as pltpu
from jax.experimental.pallas import tpu_sc as plsc

N_NODES = 10000
N_EDGES = 160000
D_FEAT = 256

NC, NS, L = 2, 16, 16
NPAD = 10240
NODES_PT = NPAD // NS
ROWS_PT = NODES_PT // 128
EH = N_EDGES // NS
EV = N_EDGES // (NC * NS)
EVP = EV + 8


def _rowsum_body(x_ref, o_ref):
    o_ref[...] = jnp.sum(x_ref[...], axis=1)[None, None, :]


def _rsqrt16(d):
    bits = lax.bitcast_convert_type(d, jnp.int32)
    y = lax.bitcast_convert_type(jnp.int32(0x5F3759DF) - (bits >> 1), jnp.float32)
    hd = 0.5 * d
    for _ in range(3):
        y = y * (1.5 - hd * y * y)
    return jnp.where(d > 0.5, y, 0.0)


def _sc_body(rowsum, eif, out,
             toh_v, ones_v, from_v, to_v, sf_v, dt_v, outb_v,
             deg_v, dis_v, s_v, rs_v, zero_v,
             hist_sh, dis_sh, s_sh, sem1, sem2):
    c = lax.axis_index("c")
    s = lax.axis_index("s")
    node0 = s * NODES_PT

    def zb(i, carry):
        zero_v[pl.ds(i * L, L)] = jnp.zeros((L,), jnp.float32)
        return carry
    lax.fori_loop(0, NODES_PT // L, zb, None)
    pltpu.sync_copy(zero_v, hist_sh.at[pl.ds(node0, NODES_PT)])
    pltpu.sync_copy(eif.at[pl.ds(N_EDGES + s * EH, EH)], toh_v)

    def ob(i, carry):
        ones_v[pl.ds(i * L, L)] = jnp.ones((L,), jnp.float32)
        return carry
    lax.fori_loop(0, EH // L, ob, None)
    plsc.subcore_barrier()

    pltpu.sync_copy(ones_v, hist_sh.at[toh_v], add=True)
    plsc.subcore_barrier()

    pltpu.sync_copy(hist_sh.at[pl.ds(node0, NODES_PT)], deg_v)
    pltpu.sync_copy(rowsum.at[pl.ds(s * ROWS_PT, ROWS_PT)], rs_v)
    for jr in range(ROWS_PT):
        def cb(ji, carry):
            w = jr * 128 + ji * L
            sl = pl.ds(w, L)
            dis = _rsqrt16(deg_v[sl])
            dis_v[sl] = dis
            s_v[sl] = dis * rs_v[jr, 0, pl.ds(ji * L, L)]
            return carry
        lax.fori_loop(0, 128 // L, cb, None)
    pltpu.sync_copy(dis_v, dis_sh.at[pl.ds(node0, NODES_PT)])
    pltpu.sync_copy(s_v, s_sh.at[pl.ds(node0, NODES_PT)])
    plsc.subcore_barrier()

    base = c * (N_EDGES // NC) + s * EV
    pltpu.sync_copy(eif.at[pl.ds(base, EV)], from_v.at[pl.ds(0, EV)])
    pltpu.sync_copy(eif.at[pl.ds(N_EDGES + base, EV)], to_v.at[pl.ds(0, EV)])
    lanes = lax.iota(jnp.int32, L)
    nvalid = EV - (EVP - L)
    tl = pl.ds(EVP - L, L)
    from_v[tl] = jnp.where(lanes < nvalid, from_v[tl], 0)
    to_v[tl] = jnp.where(lanes < nvalid, to_v[tl], 0)
    cp1 = pltpu.async_copy(s_sh.at[from_v], sf_v, sem1)
    cp2 = pltpu.async_copy(dis_sh.at[to_v], dt_v, sem2)
    cp1.wait()
    cp2.wait()

    def eb(i, carry):
        sl = pl.ds(i * L, L)
        outb_v[sl] = sf_v[sl] * dt_v[sl]
        return carry
    lax.fori_loop(0, EVP // L, eb, None)
    pltpu.sync_copy(outb_v.at[pl.ds(0, EV)], out.at[pl.ds(base, EV)])


def kernel(x, edge_index):
    x = x.astype(jnp.float32)
    eif = edge_index.astype(jnp.int32).reshape(-1)

    rowsum3d = pl.pallas_call(
        _rowsum_body,
        grid=(NPAD // 128,),
        in_specs=[pl.BlockSpec((128, D_FEAT), lambda i: (jnp.minimum(i, 78), 0))],
        out_specs=pl.BlockSpec((1, 1, 128), lambda i: (i, 0, 0)),
        out_shape=jax.ShapeDtypeStruct((NPAD // 128, 1, 128), jnp.float32),
    )(x)

    mesh = plsc.VectorSubcoreMesh(core_axis_name="c", subcore_axis_name="s")
    sc = pl.kernel(
        _sc_body,
        out_type=jax.ShapeDtypeStruct((N_EDGES,), jnp.float32),
        mesh=mesh,
        scratch_types=[
            pltpu.VMEM((EH,), jnp.int32),
            pltpu.VMEM((EH,), jnp.float32),
            pltpu.VMEM((EVP,), jnp.int32),
            pltpu.VMEM((EVP,), jnp.int32),
            pltpu.VMEM((EVP,), jnp.float32),
            pltpu.VMEM((EVP,), jnp.float32),
            pltpu.VMEM((EVP,), jnp.float32),
            pltpu.VMEM((NODES_PT,), jnp.float32),
            pltpu.VMEM((NODES_PT,), jnp.float32),
            pltpu.VMEM((NODES_PT,), jnp.float32),
            pltpu.VMEM((ROWS_PT, 1, 128), jnp.float32),
            pltpu.VMEM((NODES_PT,), jnp.float32),
            pltpu.VMEM_SHARED((NPAD,), jnp.float32),
            pltpu.VMEM_SHARED((NPAD,), jnp.float32),
            pltpu.VMEM_SHARED((NPAD,), jnp.float32),
            pltpu.SemaphoreType.DMA,
            pltpu.SemaphoreType.DMA,
        ],
    )
    return sc(rowsum3d, eif)


# speedup vs baseline: 56.8467x; 1.7487x over previous
import jax
import jax.numpy as jnp
from jax import lax
from jax.experimental import pallas as pl
from jax.experimental.pallas import tpu as pltpu
from jax.experimental.pallas import tpu_sc as plsc

N_NODES = 10000
N_EDGES = 160000
D_FEAT = 256

NC, NS, L = 2, 16, 16
NPAD = 10240
NODES_PT = NPAD // NS
ROWS_PT = NODES_PT // 128
EH = N_EDGES // NS
EV = N_EDGES // (NC * NS)
EVP = EV + 8


def _rowsum_body(x_ref, o_ref):
    o_ref[...] = jnp.sum(x_ref[...], axis=1)


def _rsqrt16(d):
    bits = lax.bitcast_convert_type(d, jnp.int32)
    y = lax.bitcast_convert_type(jnp.int32(0x5F3759DF) - (bits >> 1), jnp.float32)
    hd = 0.5 * d
    for _ in range(3):
        y = y * (1.5 - hd * y * y)
    return jnp.where(d > 0.5, y, 0.0)


def _sc_body(rowsum, eif, out,
             toh_v, ones_v, from_v, to_v, sf_v, dt_v, outb_v,
             deg_v, dis_v, s_v, rs_v, zero_v,
             hist_sh, dis_sh, s_sh, sem1, sem2):
    c = lax.axis_index("c")
    s = lax.axis_index("s")
    node0 = s * NODES_PT

    def zb(i, carry):
        zero_v[pl.ds(i * L, L)] = jnp.zeros((L,), jnp.float32)
        return carry
    lax.fori_loop(0, NODES_PT // L, zb, None)
    pltpu.sync_copy(zero_v, hist_sh.at[pl.ds(node0, NODES_PT)])
    pltpu.sync_copy(eif.at[pl.ds(N_EDGES + s * EH, EH)], toh_v)

    def ob(i, carry):
        ones_v[pl.ds(i * L, L)] = jnp.ones((L,), jnp.float32)
        return carry
    lax.fori_loop(0, EH // L, ob, None)
    plsc.subcore_barrier()

    pltpu.sync_copy(ones_v, hist_sh.at[toh_v], add=True)
    plsc.subcore_barrier()

    pltpu.sync_copy(hist_sh.at[pl.ds(node0, NODES_PT)], deg_v)
    pltpu.sync_copy(rowsum.at[pl.ds(node0, NODES_PT)], rs_v)

    def cb(i, carry):
        sl = pl.ds(i * L, L)
        dis = _rsqrt16(deg_v[sl])
        dis_v[sl] = dis
        s_v[sl] = dis * rs_v[sl]
        return carry
    lax.fori_loop(0, NODES_PT // L, cb, None)
    pltpu.sync_copy(dis_v, dis_sh.at[pl.ds(node0, NODES_PT)])
    pltpu.sync_copy(s_v, s_sh.at[pl.ds(node0, NODES_PT)])
    plsc.subcore_barrier()

    base = c * (N_EDGES // NC) + s * EV
    pltpu.sync_copy(eif.at[pl.ds(base, EV)], from_v.at[pl.ds(0, EV)])
    pltpu.sync_copy(eif.at[pl.ds(N_EDGES + base, EV)], to_v.at[pl.ds(0, EV)])
    lanes = lax.iota(jnp.int32, L)
    nvalid = EV - (EVP - L)
    tl = pl.ds(EVP - L, L)
    from_v[tl] = jnp.where(lanes < nvalid, from_v[tl], 0)
    to_v[tl] = jnp.where(lanes < nvalid, to_v[tl], 0)
    cp1 = pltpu.async_copy(s_sh.at[from_v], sf_v, sem1)
    cp2 = pltpu.async_copy(dis_sh.at[to_v], dt_v, sem2)
    cp1.wait()
    cp2.wait()

    def eb(i, carry):
        sl = pl.ds(i * L, L)
        outb_v[sl] = sf_v[sl] * dt_v[sl]
        return carry
    lax.fori_loop(0, EVP // L, eb, None)
    pltpu.sync_copy(outb_v.at[pl.ds(0, EV)], out.at[pl.ds(base, EV)])


def kernel(x, edge_index):
    x = x.astype(jnp.float32)
    eif = edge_index.astype(jnp.int32).reshape(-1)

    rowsum1d = pl.pallas_call(
        _rowsum_body,
        grid=(NPAD // 1024,),
        in_specs=[pl.BlockSpec((1024, D_FEAT), lambda i: (i, 0))],
        out_specs=pl.BlockSpec((1024,), lambda i: (i,)),
        out_shape=jax.ShapeDtypeStruct((NPAD,), jnp.float32),
    )(x)

    mesh = plsc.VectorSubcoreMesh(core_axis_name="c", subcore_axis_name="s")
    sc = pl.kernel(
        _sc_body,
        out_type=jax.ShapeDtypeStruct((N_EDGES,), jnp.float32),
        mesh=mesh,
        scratch_types=[
            pltpu.VMEM((EH,), jnp.int32),
            pltpu.VMEM((EH,), jnp.float32),
            pltpu.VMEM((EVP,), jnp.int32),
            pltpu.VMEM((EVP,), jnp.int32),
            pltpu.VMEM((EVP,), jnp.float32),
            pltpu.VMEM((EVP,), jnp.float32),
            pltpu.VMEM((EVP,), jnp.float32),
            pltpu.VMEM((NODES_PT,), jnp.float32),
            pltpu.VMEM((NODES_PT,), jnp.float32),
            pltpu.VMEM((NODES_PT,), jnp.float32),
            pltpu.VMEM((NODES_PT,), jnp.float32),
            pltpu.VMEM((NODES_PT,), jnp.float32),
            pltpu.VMEM_SHARED((NPAD,), jnp.float32),
            pltpu.VMEM_SHARED((NPAD,), jnp.float32),
            pltpu.VMEM_SHARED((NPAD,), jnp.float32),
            pltpu.SemaphoreType.DMA,
            pltpu.SemaphoreType.DMA,
        ],
    )
    return sc(rowsum1d, eif)


# speedup vs baseline: 58.9795x; 1.0375x over previous
import jax
import jax.numpy as jnp
from jax import lax
from jax.experimental import pallas as pl
from jax.experimental.pallas import tpu as pltpu
from jax.experimental.pallas import tpu_sc as plsc

N_NODES = 10000
N_EDGES = 160000
D_FEAT = 256

NC, NS, L = 2, 16, 16
NPAD = 10240
NODES_PT = NPAD // NS
ROWS_PT = NODES_PT // 128
EH = N_EDGES // NS
EV = N_EDGES // (NC * NS)
EVP = EV + 8


def _rowsum_body(x_ref, o_ref):
    o_ref[...] = jnp.sum(x_ref[...], axis=1)


def _rsqrt16(d):
    bits = lax.bitcast_convert_type(d, jnp.int32)
    y = lax.bitcast_convert_type(jnp.int32(0x5F3759DF) - (bits >> 1), jnp.float32)
    hd = 0.5 * d
    for _ in range(3):
        y = y * (1.5 - hd * y * y)
    return jnp.where(d > 0.5, y, 0.0)


def _sc_body(rowsum, eif, ones_h, zeros_h, out,
             toh_v, ones_v, from_v, to_v, sf_v, dt_v, outb_v,
             deg_v, dis_v, s_v, rs_v,
             hist_sh, dis_sh, s_sh, sem1, sem2, sem3, sem4, sem5):
    c = lax.axis_index("c")
    s = lax.axis_index("s")
    node0 = s * NODES_PT
    base = c * (N_EDGES // NC) + s * EV

    cpf = pltpu.async_copy(eif.at[pl.ds(base, EV)], from_v.at[pl.ds(0, EV)], sem3)
    cpt = pltpu.async_copy(eif.at[pl.ds(N_EDGES + base, EV)], to_v.at[pl.ds(0, EV)], sem4)
    cpo = pltpu.async_copy(ones_h, ones_v, sem5)
    pltpu.sync_copy(zeros_h.at[pl.ds(node0, NODES_PT)], hist_sh.at[pl.ds(node0, NODES_PT)])
    pltpu.sync_copy(eif.at[pl.ds(N_EDGES + s * EH, EH)], toh_v)
    cpo.wait()
    plsc.subcore_barrier()

    pltpu.sync_copy(ones_v, hist_sh.at[toh_v], add=True)
    plsc.subcore_barrier()

    pltpu.sync_copy(hist_sh.at[pl.ds(node0, NODES_PT)], deg_v)
    pltpu.sync_copy(rowsum.at[pl.ds(node0, NODES_PT)], rs_v)

    @plsc.parallel_loop(0, NODES_PT // L, 1, unroll=4)
    def cb(i):
        sl = pl.ds(i * L, L)
        dis = _rsqrt16(deg_v[sl])
        dis_v[sl] = dis
        s_v[sl] = dis * rs_v[sl]
    pltpu.sync_copy(dis_v, dis_sh.at[pl.ds(node0, NODES_PT)])
    pltpu.sync_copy(s_v, s_sh.at[pl.ds(node0, NODES_PT)])
    plsc.subcore_barrier()

    cpf.wait()
    cpt.wait()
    lanes = lax.iota(jnp.int32, L)
    nvalid = EV - (EVP - L)
    tl = pl.ds(EVP - L, L)
    from_v[tl] = jnp.where(lanes < nvalid, from_v[tl], 0)
    to_v[tl] = jnp.where(lanes < nvalid, to_v[tl], 0)
    cp1 = pltpu.async_copy(s_sh.at[from_v], sf_v, sem1)
    cp2 = pltpu.async_copy(dis_sh.at[to_v], dt_v, sem2)
    cp1.wait()
    cp2.wait()

    @plsc.parallel_loop(0, EVP // L, 1, unroll=4)
    def eb(i):
        sl = pl.ds(i * L, L)
        outb_v[sl] = sf_v[sl] * dt_v[sl]
    pltpu.sync_copy(outb_v.at[pl.ds(0, EV)], out.at[pl.ds(base, EV)])


def kernel(x, edge_index):
    x = x.astype(jnp.float32)
    eif = edge_index.astype(jnp.int32).reshape(-1)

    rowsum1d = pl.pallas_call(
        _rowsum_body,
        grid=(NPAD // 1024,),
        in_specs=[pl.BlockSpec((1024, D_FEAT), lambda i: (i, 0))],
        out_specs=pl.BlockSpec((1024,), lambda i: (i,)),
        out_shape=jax.ShapeDtypeStruct((NPAD,), jnp.float32),
    )(x)

    mesh = plsc.VectorSubcoreMesh(core_axis_name="c", subcore_axis_name="s")
    sc = pl.kernel(
        _sc_body,
        out_type=jax.ShapeDtypeStruct((N_EDGES,), jnp.float32),
        mesh=mesh,
        scratch_types=[
            pltpu.VMEM((EH,), jnp.int32),
            pltpu.VMEM((EH,), jnp.float32),
            pltpu.VMEM((EVP,), jnp.int32),
            pltpu.VMEM((EVP,), jnp.int32),
            pltpu.VMEM((EVP,), jnp.float32),
            pltpu.VMEM((EVP,), jnp.float32),
            pltpu.VMEM((EVP,), jnp.float32),
            pltpu.VMEM((NODES_PT,), jnp.float32),
            pltpu.VMEM((NODES_PT,), jnp.float32),
            pltpu.VMEM((NODES_PT,), jnp.float32),
            pltpu.VMEM((NODES_PT,), jnp.float32),
            pltpu.VMEM_SHARED((NPAD,), jnp.float32),
            pltpu.VMEM_SHARED((NPAD,), jnp.float32),
            pltpu.VMEM_SHARED((NPAD,), jnp.float32),
            pltpu.SemaphoreType.DMA,
            pltpu.SemaphoreType.DMA,
            pltpu.SemaphoreType.DMA,
            pltpu.SemaphoreType.DMA,
            pltpu.SemaphoreType.DMA,
        ],
    )
    ones_h = jnp.ones((EH,), jnp.float32)
    zeros_h = jnp.zeros((NPAD,), jnp.float32)
    return sc(rowsum1d, eif, ones_h, zeros_h)


# speedup vs baseline: 59.1248x; 1.0025x over previous
import jax
import jax.numpy as jnp
import numpy as np
from jax import lax
from jax.experimental import pallas as pl
from jax.experimental.pallas import tpu as pltpu
from jax.experimental.pallas import tpu_sc as plsc

N_NODES = 10000
N_EDGES = 160000
D_FEAT = 256

NC, NS, L = 2, 16, 16
NPAD = 10240
NODES_PT = NPAD // NS
ROWS_PT = NODES_PT // 128
EH = N_EDGES // NS
EV = N_EDGES // (NC * NS)
EVP = EV + 8


def _rowsum_body(x_ref, o_ref):
    o_ref[...] = jnp.dot(x_ref[...], jnp.ones((D_FEAT,), jnp.float32),
                         preferred_element_type=jnp.float32)


def _rsqrt16(d):
    bits = lax.bitcast_convert_type(d, jnp.int32)
    y = lax.bitcast_convert_type(jnp.int32(0x5F3759DF) - (bits >> 1), jnp.float32)
    hd = 0.5 * d
    for _ in range(3):
        y = y * (1.5 - hd * y * y)
    return jnp.where(d > 0.5, y, 0.0)


def _sc_body(rowsum, eif, ones_h, zeros_h, out,
             toh_v, ones_v, from_v, to_v, sf_v, dt_v, outb_v,
             deg_v, dis_v, s_v, rs_v,
             hist_sh, dis_sh, s_sh, sem1, sem2, sem3, sem4, sem5):
    c = lax.axis_index("c")
    s = lax.axis_index("s")
    node0 = s * NODES_PT
    base = c * (N_EDGES // NC) + s * EV

    cpf = pltpu.async_copy(eif.at[pl.ds(base, EV)], from_v.at[pl.ds(0, EV)], sem3)
    cpt = pltpu.async_copy(eif.at[pl.ds(N_EDGES + base, EV)], to_v.at[pl.ds(0, EV)], sem4)
    cpo = pltpu.async_copy(ones_h, ones_v, sem5)
    pltpu.sync_copy(zeros_h.at[pl.ds(node0, NODES_PT)], hist_sh.at[pl.ds(node0, NODES_PT)])
    pltpu.sync_copy(eif.at[pl.ds(N_EDGES + s * EH, EH)], toh_v)
    cpo.wait()
    plsc.subcore_barrier()

    pltpu.sync_copy(ones_v, hist_sh.at[toh_v], add=True)
    plsc.subcore_barrier()

    pltpu.sync_copy(hist_sh.at[pl.ds(node0, NODES_PT)], deg_v)
    pltpu.sync_copy(rowsum.at[pl.ds(node0, NODES_PT)], rs_v)

    @plsc.parallel_loop(0, NODES_PT // L, 1, unroll=4)
    def cb(i):
        sl = pl.ds(i * L, L)
        dis = _rsqrt16(deg_v[sl])
        dis_v[sl] = dis
        s_v[sl] = dis * rs_v[sl]
    pltpu.sync_copy(dis_v, dis_sh.at[pl.ds(node0, NODES_PT)])
    pltpu.sync_copy(s_v, s_sh.at[pl.ds(node0, NODES_PT)])
    plsc.subcore_barrier()

    cpf.wait()
    cpt.wait()
    lanes = lax.iota(jnp.int32, L)
    nvalid = EV - (EVP - L)
    tl = pl.ds(EVP - L, L)
    from_v[tl] = jnp.where(lanes < nvalid, from_v[tl], 0)
    to_v[tl] = jnp.where(lanes < nvalid, to_v[tl], 0)
    cp1 = pltpu.async_copy(s_sh.at[from_v], sf_v, sem1)
    cp2 = pltpu.async_copy(dis_sh.at[to_v], dt_v, sem2)
    cp1.wait()
    cp2.wait()

    @plsc.parallel_loop(0, EVP // L, 1, unroll=4)
    def eb(i):
        sl = pl.ds(i * L, L)
        outb_v[sl] = sf_v[sl] * dt_v[sl]
    pltpu.sync_copy(outb_v.at[pl.ds(0, EV)], out.at[pl.ds(base, EV)])


def kernel(x, edge_index):
    x = x.astype(jnp.float32)
    eif = edge_index.astype(jnp.int32).reshape(-1)

    rowsum1d = pl.pallas_call(
        _rowsum_body,
        grid=(NPAD // 1024,),
        in_specs=[pl.BlockSpec((1024, D_FEAT), lambda i: (i, 0))],
        out_specs=pl.BlockSpec((1024,), lambda i: (i,)),
        out_shape=jax.ShapeDtypeStruct((NPAD,), jnp.float32),
    )(x)

    mesh = plsc.VectorSubcoreMesh(core_axis_name="c", subcore_axis_name="s")
    sc = pl.kernel(
        _sc_body,
        out_type=jax.ShapeDtypeStruct((N_EDGES,), jnp.float32),
        mesh=mesh,
        scratch_types=[
            pltpu.VMEM((EH,), jnp.int32),
            pltpu.VMEM((EH,), jnp.float32),
            pltpu.VMEM((EVP,), jnp.int32),
            pltpu.VMEM((EVP,), jnp.int32),
            pltpu.VMEM((EVP,), jnp.float32),
            pltpu.VMEM((EVP,), jnp.float32),
            pltpu.VMEM((EVP,), jnp.float32),
            pltpu.VMEM((NODES_PT,), jnp.float32),
            pltpu.VMEM((NODES_PT,), jnp.float32),
            pltpu.VMEM((NODES_PT,), jnp.float32),
            pltpu.VMEM((NODES_PT,), jnp.float32),
            pltpu.VMEM_SHARED((NPAD,), jnp.float32),
            pltpu.VMEM_SHARED((NPAD,), jnp.float32),
            pltpu.VMEM_SHARED((NPAD,), jnp.float32),
            pltpu.SemaphoreType.DMA,
            pltpu.SemaphoreType.DMA,
            pltpu.SemaphoreType.DMA,
            pltpu.SemaphoreType.DMA,
            pltpu.SemaphoreType.DMA,
        ],
    )
    ones_h = jnp.asarray(np.ones((EH,), np.float32))
    zeros_h = jnp.asarray(np.zeros((NPAD,), np.float32))
    return sc(rowsum1d, eif, ones_h, zeros_h)


# speedup vs baseline: 67.8168x; 1.1470x over previous
import jax
import jax.numpy as jnp
from jax import lax
from jax.experimental import pallas as pl
from jax.experimental.pallas import tpu as pltpu
from jax.experimental.pallas import tpu_sc as plsc

N_NODES = 10000
N_EDGES = 160000
D_FEAT = 256

NC, NS, L = 2, 16, 16
NPAD = 10240
NODES_PT = NPAD // NS
EH = N_EDGES // NS
EV = N_EDGES // (NC * NS)
EVP = EV + 8


def _rowsum_body(x_ref, o_ref):
    o_ref[...] = jnp.dot(x_ref[...], jnp.ones((D_FEAT,), jnp.float32),
                         preferred_element_type=jnp.float32)


def _rsqrt16(d):
    bits = lax.bitcast_convert_type(d, jnp.int32)
    y = lax.bitcast_convert_type(jnp.int32(0x5F3759DF) - (bits >> 1), jnp.float32)
    hd = 0.5 * d
    for _ in range(3):
        y = y * (1.5 - hd * y * y)
    return jnp.where(d > 0.5, y, 0.0)


def _sc_body(rowsum, eif, out,
             toh_v, ones_v, from_v, to_v, sf_v, dt_v, outb_v,
             deg_v, dis_v, s_v, rs_v, zero_v,
             hist_sh, dis_sh, s_sh, sem1, sem2, sem3, sem4, sem5):
    c = lax.axis_index("c")
    s = lax.axis_index("s")
    node0 = s * NODES_PT
    base = c * (N_EDGES // NC) + s * EV

    cpf = pltpu.async_copy(eif.at[pl.ds(base, EV)], from_v.at[pl.ds(0, EV)], sem3)
    cpt = pltpu.async_copy(eif.at[pl.ds(N_EDGES + base, EV)], to_v.at[pl.ds(0, EV)], sem4)
    cph = pltpu.async_copy(eif.at[pl.ds(N_EDGES + s * EH, EH)], toh_v, sem5)

    @plsc.parallel_loop(0, EH // L, 1, unroll=4)
    def ob(i):
        ones_v[pl.ds(i * L, L)] = jnp.ones((L,), jnp.float32)

    @plsc.parallel_loop(0, NODES_PT // L, 1, unroll=4)
    def zb(i):
        zero_v[pl.ds(i * L, L)] = jnp.zeros((L,), jnp.float32)
    pltpu.sync_copy(zero_v, hist_sh.at[pl.ds(node0, NODES_PT)])
    cph.wait()
    plsc.subcore_barrier()

    pltpu.sync_copy(ones_v, hist_sh.at[toh_v], add=True)
    plsc.subcore_barrier()

    pltpu.sync_copy(hist_sh.at[pl.ds(node0, NODES_PT)], deg_v)
    pltpu.sync_copy(rowsum.at[pl.ds(node0, NODES_PT)], rs_v)

    @plsc.parallel_loop(0, NODES_PT // L, 1, unroll=4)
    def cb(i):
        sl = pl.ds(i * L, L)
        dis = _rsqrt16(deg_v[sl])
        dis_v[sl] = dis
        s_v[sl] = dis * rs_v[sl]
    pltpu.sync_copy(dis_v, dis_sh.at[pl.ds(node0, NODES_PT)])
    pltpu.sync_copy(s_v, s_sh.at[pl.ds(node0, NODES_PT)])
    plsc.subcore_barrier()

    cpf.wait()
    cpt.wait()
    lanes = lax.iota(jnp.int32, L)
    nvalid = EV - (EVP - L)
    tl = pl.ds(EVP - L, L)
    from_v[tl] = jnp.where(lanes < nvalid, from_v[tl], 0)
    to_v[tl] = jnp.where(lanes < nvalid, to_v[tl], 0)
    cp1 = pltpu.async_copy(s_sh.at[from_v], sf_v, sem1)
    cp2 = pltpu.async_copy(dis_sh.at[to_v], dt_v, sem2)
    cp1.wait()
    cp2.wait()

    @plsc.parallel_loop(0, EVP // L, 1, unroll=4)
    def eb(i):
        sl = pl.ds(i * L, L)
        outb_v[sl] = sf_v[sl] * dt_v[sl]
    pltpu.sync_copy(outb_v.at[pl.ds(0, EV)], out.at[pl.ds(base, EV)])


def kernel(x, edge_index):
    x = x.astype(jnp.float32)
    eif = edge_index.astype(jnp.int32).reshape(-1)

    rowsum1d = pl.pallas_call(
        _rowsum_body,
        grid=(NPAD // 2048,),
        in_specs=[pl.BlockSpec((2048, D_FEAT), lambda i: (i, 0))],
        out_specs=pl.BlockSpec((2048,), lambda i: (i,)),
        out_shape=jax.ShapeDtypeStruct((NPAD,), jnp.float32),
    )(x)

    mesh = plsc.VectorSubcoreMesh(core_axis_name="c", subcore_axis_name="s")
    sc = pl.kernel(
        _sc_body,
        out_type=jax.ShapeDtypeStruct((N_EDGES,), jnp.float32),
        mesh=mesh,
        scratch_types=[
            pltpu.VMEM((EH,), jnp.int32),
            pltpu.VMEM((EH,), jnp.float32),
            pltpu.VMEM((EVP,), jnp.int32),
            pltpu.VMEM((EVP,), jnp.int32),
            pltpu.VMEM((EVP,), jnp.float32),
            pltpu.VMEM((EVP,), jnp.float32),
            pltpu.VMEM((EVP,), jnp.float32),
            pltpu.VMEM((NODES_PT,), jnp.float32),
            pltpu.VMEM((NODES_PT,), jnp.float32),
            pltpu.VMEM((NODES_PT,), jnp.float32),
            pltpu.VMEM((NODES_PT,), jnp.float32),
            pltpu.VMEM((NODES_PT,), jnp.float32),
            pltpu.VMEM_SHARED((NPAD,), jnp.float32),
            pltpu.VMEM_SHARED((NPAD,), jnp.float32),
            pltpu.VMEM_SHARED((NPAD,), jnp.float32),
            pltpu.SemaphoreType.DMA,
            pltpu.SemaphoreType.DMA,
            pltpu.SemaphoreType.DMA,
            pltpu.SemaphoreType.DMA,
            pltpu.SemaphoreType.DMA,
        ],
    )
    return sc(rowsum1d, eif)


# speedup vs baseline: 72.8520x; 1.0742x over previous
import jax
import jax.numpy as jnp
from jax import lax
from jax.experimental import pallas as pl
from jax.experimental.pallas import tpu as pltpu
from jax.experimental.pallas import tpu_sc as plsc

N_NODES = 10000
N_EDGES = 160000
D_FEAT = 256

NC, NS, L = 2, 16, 16
NPAD = 10240
NODES_PT = NPAD // NS
NODES_HC = NODES_PT // NC
EH = N_EDGES // NS
EV = N_EDGES // (NC * NS)
EVP = EV + 8


def _rowsum_body(x_ref, o_ref):
    o_ref[...] = jnp.dot(x_ref[...], jnp.ones((D_FEAT,), jnp.float32),
                         preferred_element_type=jnp.float32)


def _rsqrt16(d):
    bits = lax.bitcast_convert_type(d, jnp.int32)
    y = lax.bitcast_convert_type(jnp.int32(0x5F3759DF) - (bits >> 1), jnp.float32)
    hd = 0.5 * d
    for _ in range(3):
        y = y * (1.5 - hd * y * y)
    return jnp.where(d > 0.5, y, 0.0)


def _sc_hist_body(eif, dish,
                  toh_v, ones_v, deg_v, dis_v, zero_v, hist_sh, sem5):
    c = lax.axis_index("c")
    s = lax.axis_index("s")
    node0 = s * NODES_PT

    with jax.named_scope("phA"):
        cph = pltpu.async_copy(eif.at[pl.ds(N_EDGES + s * EH, EH)], toh_v, sem5)

        @plsc.parallel_loop(0, EH // L, 1, unroll=4)
        def ob(i):
            ones_v[pl.ds(i * L, L)] = jnp.ones((L,), jnp.float32)

        @plsc.parallel_loop(0, NODES_PT // L, 1, unroll=4)
        def zb(i):
            zero_v[pl.ds(i * L, L)] = jnp.zeros((L,), jnp.float32)
        pltpu.sync_copy(zero_v, hist_sh.at[pl.ds(node0, NODES_PT)])
        cph.wait()
        plsc.subcore_barrier()

    with jax.named_scope("phB"):
        pltpu.sync_copy(ones_v, hist_sh.at[toh_v], add=True)
        plsc.subcore_barrier()

    with jax.named_scope("phDis"):
        pltpu.sync_copy(hist_sh.at[pl.ds(node0, NODES_PT)], deg_v)

        @plsc.parallel_loop(0, NODES_PT // L, 1, unroll=4)
        def cb(i):
            sl = pl.ds(i * L, L)
            dis_v[sl] = _rsqrt16(deg_v[sl])
        half = c * NODES_HC
        pltpu.sync_copy(dis_v.at[pl.ds(half, NODES_HC)],
                        dish.at[pl.ds(node0 + half, NODES_HC)])


def _sc_prop_body(rowsum, eif, dish, out,
                  from_v, to_v, sf_v, dt_v, outb_v,
                  dis_v, s_v, rs_v,
                  dis_sh, s_sh, sem1, sem2, sem3, sem4):
    c = lax.axis_index("c")
    s = lax.axis_index("s")
    node0 = s * NODES_PT
    base = c * (N_EDGES // NC) + s * EV

    with jax.named_scope("phC"):
        cpf = pltpu.async_copy(eif.at[pl.ds(base, EV)], from_v.at[pl.ds(0, EV)], sem3)
        cpt = pltpu.async_copy(eif.at[pl.ds(N_EDGES + base, EV)], to_v.at[pl.ds(0, EV)], sem4)
        pltpu.sync_copy(dish.at[pl.ds(node0, NODES_PT)], dis_v)
        pltpu.sync_copy(rowsum.at[pl.ds(node0, NODES_PT)], rs_v)

        @plsc.parallel_loop(0, NODES_PT // L, 1, unroll=4)
        def cb(i):
            sl = pl.ds(i * L, L)
            s_v[sl] = dis_v[sl] * rs_v[sl]
        pltpu.sync_copy(dis_v, dis_sh.at[pl.ds(node0, NODES_PT)])
        pltpu.sync_copy(s_v, s_sh.at[pl.ds(node0, NODES_PT)])
        plsc.subcore_barrier()

    with jax.named_scope("phD"):
        cpf.wait()
        cpt.wait()
        lanes = lax.iota(jnp.int32, L)
        nvalid = EV - (EVP - L)
        tl = pl.ds(EVP - L, L)
        from_v[tl] = jnp.where(lanes < nvalid, from_v[tl], 0)
        to_v[tl] = jnp.where(lanes < nvalid, to_v[tl], 0)
        cp1 = pltpu.async_copy(s_sh.at[from_v], sf_v, sem1)
        cp2 = pltpu.async_copy(dis_sh.at[to_v], dt_v, sem2)
        cp1.wait()
        cp2.wait()

        @plsc.parallel_loop(0, EVP // L, 1, unroll=4)
        def eb(i):
            sl = pl.ds(i * L, L)
            outb_v[sl] = sf_v[sl] * dt_v[sl]
        pltpu.sync_copy(outb_v.at[pl.ds(0, EV)], out.at[pl.ds(base, EV)])


def kernel(x, edge_index):
    x = x.astype(jnp.float32)
    eif = edge_index.astype(jnp.int32).reshape(-1)

    mesh = plsc.VectorSubcoreMesh(core_axis_name="c", subcore_axis_name="s")
    sc_hist = pl.kernel(
        _sc_hist_body,
        out_type=jax.ShapeDtypeStruct((NPAD,), jnp.float32),
        mesh=mesh,
        scratch_types=[
            pltpu.VMEM((EH,), jnp.int32),
            pltpu.VMEM((EH,), jnp.float32),
            pltpu.VMEM((NODES_PT,), jnp.float32),
            pltpu.VMEM((NODES_PT,), jnp.float32),
            pltpu.VMEM((NODES_PT,), jnp.float32),
            pltpu.VMEM_SHARED((NPAD,), jnp.float32),
            pltpu.SemaphoreType.DMA,
        ],
    )
    dish = sc_hist(eif)

    rowsum1d = pl.pallas_call(
        _rowsum_body,
        grid=(NPAD // 2048,),
        in_specs=[pl.BlockSpec((2048, D_FEAT), lambda i: (i, 0))],
        out_specs=pl.BlockSpec((2048,), lambda i: (i,)),
        out_shape=jax.ShapeDtypeStruct((NPAD,), jnp.float32),
    )(x)

    sc_prop = pl.kernel(
        _sc_prop_body,
        out_type=jax.ShapeDtypeStruct((N_EDGES,), jnp.float32),
        mesh=mesh,
        scratch_types=[
            pltpu.VMEM((EVP,), jnp.int32),
            pltpu.VMEM((EVP,), jnp.int32),
            pltpu.VMEM((EVP,), jnp.float32),
            pltpu.VMEM((EVP,), jnp.float32),
            pltpu.VMEM((EVP,), jnp.float32),
            pltpu.VMEM((NODES_PT,), jnp.float32),
            pltpu.VMEM((NODES_PT,), jnp.float32),
            pltpu.VMEM((NODES_PT,), jnp.float32),
            pltpu.VMEM_SHARED((NPAD,), jnp.float32),
            pltpu.VMEM_SHARED((NPAD,), jnp.float32),
            pltpu.SemaphoreType.DMA,
            pltpu.SemaphoreType.DMA,
            pltpu.SemaphoreType.DMA,
            pltpu.SemaphoreType.DMA,
        ],
    )
    return sc_prop(rowsum1d, eif, dish)
